# baseline (speedup 1.0000x reference)
import functools

import jax
import jax.numpy as jnp
from jax import lax
from jax.experimental import pallas as pl
from jax.experimental.pallas import tpu as pltpu
from jax.experimental.pallas import tpu_sc as plsc

_DIM = 1024
_INTER = 512
_E = 8
_TOPK = 2
_SHINTER = 1024
_TILE = 256
_T = 2048
_NP = _TOPK * _T
_PADT = _NP + _E * _TILE
_NRT = _PADT // _TILE
_NST = _T // _TILE
_ZBASE = _PADT
_NW = 32


def _dot_t(a, b, prec=None):
    return jax.lax.dot_general(
        a, b, (((1,), (1,)), ((), ())),
        preferred_element_type=jnp.float32, precision=prec)


def _gate_body(xb_ref, wg_ref, am_ref, w01_ref, l_ref):
    T = xb_ref.shape[0]
    scores = _dot_t(xb_ref[...], wg_ref[...])
    p = jax.nn.sigmoid(scores)
    iota = jax.lax.broadcasted_iota(jnp.int32, p.shape, 1)
    m1 = jnp.max(p, axis=1, keepdims=True)
    am1 = jnp.min(jnp.where(p == m1, iota, _E), axis=1, keepdims=True)
    p2 = jnp.where(iota == am1, -1.0, p)
    m2 = jnp.max(p2, axis=1, keepdims=True)
    am2 = jnp.min(jnp.where(p2 == m2, iota, _E), axis=1, keepdims=True)
    s = m1 + m2
    am_ref[...] = jnp.concatenate([am1, am2], axis=1)
    w01_ref[...] = jnp.concatenate([m1 / s, m2 / s], axis=1)
    w = (jnp.where(iota == am1, m1, 0.0) +
         jnp.where(iota == am2, m2, 0.0)) / s
    sel = ((iota == am1) | (iota == am2)).astype(jnp.float32)
    counts = jnp.sum(sel, axis=0, keepdims=True)
    probs = jnp.sum(w, axis=0, keepdims=True)
    f_i = _E * counts / (_TOPK * T)
    p_i = probs / T
    l_ref[...] = jnp.sum(f_i * p_i, axis=1, keepdims=True)


def _gate(xb, Wgb):
    return pl.pallas_call(
        _gate_body,
        out_shape=[
            jax.ShapeDtypeStruct((_T, 2), jnp.int32),
            jax.ShapeDtypeStruct((_T, 2), jnp.float32),
            jax.ShapeDtypeStruct((1, 1), jnp.float32),
        ],
    )(xb, Wgb)


def _sc_gather(xf3, tok_padded):
    rows_per = _PADT // _NW
    half = rows_per // 2
    mesh = plsc.VectorSubcoreMesh(core_axis_name="c", subcore_axis_name="s")

    @functools.partial(
        pl.kernel, mesh=mesh,
        out_type=jax.ShapeDtypeStruct((_PADT, 8, 128), jnp.float32),
        scratch_types=[
            pltpu.VMEM((2, half), jnp.int32),
            pltpu.VMEM((half, 8, 128), jnp.float32),
            pltpu.SemaphoreType.DMA,
        ],
    )
    def k(xf_hbm, tok_hbm, xs_hbm, idx_v, rows_v, sem):
        wid = lax.axis_index("s") * 2 + lax.axis_index("c")
        base = wid * rows_per
        pltpu.sync_copy(tok_hbm.at[pl.ds(2 * wid, 2)], idx_v)
        for i in range(2):
            pltpu.async_copy(xf_hbm.at[idx_v.at[i]], rows_v, sem).wait()
            pltpu.sync_copy(rows_v, xs_hbm.at[pl.ds(base + i * half, half)])

    return k(xf3, tok_padded)


def _grouped_body(eid_ref, xs_ref, xb_ref, wp_ref, w13_ref, w2_ref,
                  ws13_ref, bs13_ref, ws2_ref, bs2_ref, out_ref):
    i = pl.program_id(0)

    @pl.when(i < _NRT)
    def _routed():
        g = _dot_t(xs_ref[...].astype(jnp.bfloat16), w13_ref[0])
        h = jax.nn.silu(g[:, :_INTER]) * g[:, _INTER:] * wp_ref[...]
        out_ref[...] = _dot_t(h.astype(jnp.bfloat16), w2_ref[0])

    @pl.when(i >= _NRT)
    def _shared():
        gs = _dot_t(xb_ref[...], ws13_ref[...]) + bs13_ref[...]
        hs = jax.nn.silu(gs[:, :_SHINTER]) * gs[:, _SHINTER:]
        out_ref[...] = _dot_t(hs.astype(jnp.bfloat16),
                              ws2_ref[...]) + bs2_ref[...]


def _grouped(eid_all, xs2, xb, wp, W13, W2b, Ws13, bs13, Ws2b, bs2r):
    grid_spec = pltpu.PrefetchScalarGridSpec(
        num_scalar_prefetch=1,
        grid=(_NRT + _NST,),
        in_specs=[
            pl.BlockSpec((_TILE, _DIM),
                         lambda i, eid: (jnp.minimum(i, _NRT - 1), 0)),
            pl.BlockSpec((_TILE, _DIM),
                         lambda i, eid: (jnp.maximum(i - _NRT, 0), 0)),
            pl.BlockSpec((_TILE, 1),
                         lambda i, eid: (jnp.minimum(i, _NRT - 1), 0)),
            pl.BlockSpec((1, 2 * _INTER, _DIM),
                         lambda i, eid: (jnp.minimum(eid[i], _E - 1), 0, 0)),
            pl.BlockSpec((1, _DIM, _INTER),
                         lambda i, eid: (jnp.minimum(eid[i], _E - 1), 0, 0)),
            pl.BlockSpec((2 * _SHINTER, _DIM), lambda i, eid: (0, 0)),
            pl.BlockSpec((1, 2 * _SHINTER), lambda i, eid: (0, 0)),
            pl.BlockSpec((_DIM, _SHINTER), lambda i, eid: (0, 0)),
            pl.BlockSpec((1, _DIM), lambda i, eid: (0, 0)),
        ],
        out_specs=pl.BlockSpec((_TILE, _DIM), lambda i, eid: (i, 0)),
    )
    return pl.pallas_call(
        _grouped_body,
        grid_spec=grid_spec,
        out_shape=jax.ShapeDtypeStruct((_PADT + _T, _DIM), jnp.float32),
        compiler_params=pltpu.CompilerParams(
            dimension_semantics=("arbitrary",)),
    )(eid_all, xs2, xb, wp, W13, W2b, Ws13, bs13, Ws2b, bs2r)


def _sc_combine(out3, p0, p1):
    tok_per = _T // _NW
    chunk = 16
    nchunk = tok_per // chunk
    mesh = plsc.VectorSubcoreMesh(core_axis_name="c", subcore_axis_name="s")

    @functools.partial(
        pl.kernel, mesh=mesh,
        out_type=jax.ShapeDtypeStruct((_T, 8, 128), jnp.float32),
        scratch_types=[
            pltpu.VMEM((chunk,), jnp.int32),
            pltpu.VMEM((chunk,), jnp.int32),
            pltpu.VMEM((chunk, 8, 128), jnp.float32),
            pltpu.VMEM((chunk, 8, 128), jnp.float32),
            pltpu.VMEM((chunk, 8, 128), jnp.float32),
            pltpu.VMEM((chunk, 8, 128), jnp.float32),
            pltpu.SemaphoreType.DMA,
        ],
    )
    def k(out_hbm, p0_hbm, p1_hbm, y_hbm,
          i0_v, i1_v, r0_v, r1_v, rz_v, y_v, sem):
        wid = lax.axis_index("s") * 2 + lax.axis_index("c")

        def body(c, carry):
            base = wid * tok_per + c * chunk
            pltpu.sync_copy(p0_hbm.at[pl.ds(base, chunk)], i0_v)
            pltpu.sync_copy(p1_hbm.at[pl.ds(base, chunk)], i1_v)
            pltpu.async_copy(out_hbm.at[i0_v], r0_v, sem).wait()
            pltpu.async_copy(out_hbm.at[i1_v], r1_v, sem).wait()
            pltpu.sync_copy(out_hbm.at[pl.ds(_ZBASE + base, chunk)], rz_v)
            for r in range(chunk):
                for s in range(8):
                    for q in range(8):
                        sl = pl.ds(q * 16, 16)
                        y_v[r, s, sl] = (r0_v[r, s, sl] + r1_v[r, s, sl]
                                         + rz_v[r, s, sl])
            pltpu.sync_copy(y_v, y_hbm.at[pl.ds(base, chunk)])
            return carry

        lax.fori_loop(0, nchunk, body, 0)

    return k(out3, p0, p1)


def kernel(x, Wg, W1, W2, W3, Ws1, bs1, Ws2, bs2, Ws3, bs3):
    orig_shape = x.shape
    xf = x.reshape(-1, _DIM)
    bf = jnp.bfloat16
    xb = xf.astype(bf)
    Wgb = Wg.astype(bf)
    W13 = jnp.concatenate([W1, W3], axis=1).astype(bf)
    W2b = W2.astype(bf)
    Ws13 = jnp.concatenate([Ws1, Ws3], axis=0).astype(bf)
    bs13 = jnp.concatenate([bs1, bs3]).reshape(1, 2 * _SHINTER)
    Ws2b = Ws2.astype(bf)
    bs2r = bs2.reshape(1, _DIM)

    am, w01, l = _gate(xb, Wgb)

    e_flat = am.reshape(-1)
    oh = (e_flat[:, None] ==
          jnp.arange(_E, dtype=jnp.int32)[None, :]).astype(jnp.int32)
    cc = jnp.cumsum(oh, axis=0)
    counts = cc[-1]
    rank = jnp.take_along_axis(cc, e_flat[:, None], axis=1)[:, 0] - 1
    pc = ((counts + _TILE - 1) // _TILE) * _TILE
    ends = jnp.cumsum(pc)
    pofs = ends - pc
    p_flat = pofs[e_flat] + rank
    tile_eid = jnp.minimum(
        jnp.searchsorted(ends, jnp.arange(_NRT, dtype=jnp.int32) * _TILE,
                         side='right'),
        _E - 1).astype(jnp.int32)
    eid_all = jnp.concatenate(
        [tile_eid, jnp.full((_NST,), _E, jnp.int32)])
    tok = jnp.arange(_NP, dtype=jnp.int32) // 2
    tok_padded = jnp.zeros((_PADT,), jnp.int32).at[p_flat].set(tok)
    w_padded = jnp.zeros((_PADT,), jnp.float32).at[p_flat].set(
        w01.reshape(-1))
    p01 = p_flat.reshape(_T, 2)

    xs3 = _sc_gather(xf.reshape(_T, 8, 128),
                     tok_padded.reshape(2 * _NW, _PADT // (2 * _NW)))

    out = _grouped(eid_all, xs3.reshape(_PADT, _DIM), xb,
                   w_padded.reshape(_PADT, 1), W13, W2b,
                   Ws13, bs13, Ws2b, bs2r)

    y3 = _sc_combine(out.reshape(_PADT + _T, 8, 128),
                     p01[:, 0], p01[:, 1])

    return y3.reshape(orig_shape), l[0, 0]


# speedup vs baseline: 1.1271x; 1.1271x over previous
import functools

import jax
import jax.numpy as jnp
from jax import lax
from jax.experimental import pallas as pl
from jax.experimental.pallas import tpu as pltpu
from jax.experimental.pallas import tpu_sc as plsc

_DIM = 1024
_INTER = 512
_E = 8
_TOPK = 2
_SHINTER = 1024
_TILE = 256
_T = 2048
_NP = _TOPK * _T
_PADT = _NP + _E * _TILE
_NRT = _PADT // _TILE
_NST = _T // _TILE
_ZBASE = _PADT
_NW = 32


def _dot_t(a, b, prec=None):
    return jax.lax.dot_general(
        a, b, (((1,), (1,)), ((), ())),
        preferred_element_type=jnp.float32, precision=prec)


def _gate_body(xb_ref, wg_ref, am_ref, w01_ref, l_ref):
    T = xb_ref.shape[0]
    scores = _dot_t(xb_ref[...], wg_ref[...])
    p = jax.nn.sigmoid(scores)
    iota = jax.lax.broadcasted_iota(jnp.int32, p.shape, 1)
    m1 = jnp.max(p, axis=1, keepdims=True)
    am1 = jnp.min(jnp.where(p == m1, iota, _E), axis=1, keepdims=True)
    p2 = jnp.where(iota == am1, -1.0, p)
    m2 = jnp.max(p2, axis=1, keepdims=True)
    am2 = jnp.min(jnp.where(p2 == m2, iota, _E), axis=1, keepdims=True)
    s = m1 + m2
    am_ref[...] = jnp.concatenate([am1, am2], axis=1)
    w01_ref[...] = jnp.concatenate([m1 / s, m2 / s], axis=1)
    w = (jnp.where(iota == am1, m1, 0.0) +
         jnp.where(iota == am2, m2, 0.0)) / s
    sel = ((iota == am1) | (iota == am2)).astype(jnp.float32)
    counts = jnp.sum(sel, axis=0, keepdims=True)
    probs = jnp.sum(w, axis=0, keepdims=True)
    f_i = _E * counts / (_TOPK * T)
    p_i = probs / T
    l_ref[...] = jnp.sum(f_i * p_i, axis=1, keepdims=True)


def _gate(xb, Wgb):
    return pl.pallas_call(
        _gate_body,
        out_shape=[
            jax.ShapeDtypeStruct((_T, 2), jnp.int32),
            jax.ShapeDtypeStruct((_T, 2), jnp.float32),
            jax.ShapeDtypeStruct((1, 1), jnp.float32),
        ],
    )(xb, Wgb)


def _sc_gather(xf2, tok_padded):
    rows_per = _PADT // _NW
    nch = 4
    ch = rows_per // nch
    mesh = plsc.VectorSubcoreMesh(core_axis_name="c", subcore_axis_name="s")

    @functools.partial(
        pl.kernel, mesh=mesh,
        out_type=jax.ShapeDtypeStruct((_PADT, _DIM), jnp.float32),
        scratch_types=[
            pltpu.VMEM((nch, ch), jnp.int32),
            pltpu.VMEM((ch, _DIM), jnp.float32),
            pltpu.VMEM((ch, _DIM), jnp.float32),
            pltpu.SemaphoreType.DMA,
            pltpu.SemaphoreType.DMA,
            pltpu.SemaphoreType.DMA,
            pltpu.SemaphoreType.DMA,
        ],
    )
    def k(xf_hbm, tok_hbm, xs_hbm, idx_v, r0_v, r1_v, g0, g1, w0, w1):
        wid = lax.axis_index("s") * 2 + lax.axis_index("c")
        base = wid * rows_per
        pltpu.sync_copy(tok_hbm.at[pl.ds(nch * wid, nch)], idx_v)
        bufs = (r0_v, r1_v)
        gsems = (g0, g1)
        wsems = (w0, w1)
        gets = [None] * nch
        puts = [None] * nch
        for i in range(nch):
            if i >= 2:
                puts[i - 2].wait()
            gets[i] = pltpu.async_copy(
                xf_hbm.at[idx_v.at[i]], bufs[i % 2], gsems[i % 2])
            if i >= 1:
                gets[i - 1].wait()
                puts[i - 1] = pltpu.async_copy(
                    bufs[(i - 1) % 2],
                    xs_hbm.at[pl.ds(base + (i - 1) * ch, ch)],
                    wsems[(i - 1) % 2])
        gets[nch - 1].wait()
        puts[nch - 1] = pltpu.async_copy(
            bufs[(nch - 1) % 2],
            xs_hbm.at[pl.ds(base + (nch - 1) * ch, ch)],
            wsems[(nch - 1) % 2])
        puts[nch - 2].wait()
        puts[nch - 1].wait()

    return k(xf2, tok_padded)


def _grouped_body(eid_ref, xs_ref, xb_ref, wp_ref, w13_ref, w2_ref,
                  ws13_ref, bs13_ref, ws2_ref, bs2_ref, out_ref):
    i = pl.program_id(0)

    @pl.when(i < _NRT)
    def _routed():
        g = _dot_t(xs_ref[...].astype(jnp.bfloat16), w13_ref[0])
        h = jax.nn.silu(g[:, :_INTER]) * g[:, _INTER:] * wp_ref[...]
        out_ref[...] = _dot_t(h.astype(jnp.bfloat16), w2_ref[0])

    @pl.when(i >= _NRT)
    def _shared():
        gs = _dot_t(xb_ref[...], ws13_ref[...]) + bs13_ref[...]
        hs = jax.nn.silu(gs[:, :_SHINTER]) * gs[:, _SHINTER:]
        out_ref[...] = _dot_t(hs.astype(jnp.bfloat16),
                              ws2_ref[...]) + bs2_ref[...]


def _grouped(eid_all, xs2, xb, wp, W13, W2b, Ws13, bs13, Ws2b, bs2r):
    grid_spec = pltpu.PrefetchScalarGridSpec(
        num_scalar_prefetch=1,
        grid=(_NRT + _NST,),
        in_specs=[
            pl.BlockSpec((_TILE, _DIM),
                         lambda i, eid: (jnp.minimum(i, _NRT - 1), 0)),
            pl.BlockSpec((_TILE, _DIM),
                         lambda i, eid: (jnp.maximum(i - _NRT, 0), 0)),
            pl.BlockSpec((_TILE, 1),
                         lambda i, eid: (jnp.minimum(i, _NRT - 1), 0)),
            pl.BlockSpec((1, 2 * _INTER, _DIM),
                         lambda i, eid: (jnp.minimum(eid[i], _E - 1), 0, 0)),
            pl.BlockSpec((1, _DIM, _INTER),
                         lambda i, eid: (jnp.minimum(eid[i], _E - 1), 0, 0)),
            pl.BlockSpec((2 * _SHINTER, _DIM), lambda i, eid: (0, 0)),
            pl.BlockSpec((1, 2 * _SHINTER), lambda i, eid: (0, 0)),
            pl.BlockSpec((_DIM, _SHINTER), lambda i, eid: (0, 0)),
            pl.BlockSpec((1, _DIM), lambda i, eid: (0, 0)),
        ],
        out_specs=pl.BlockSpec((_TILE, _DIM), lambda i, eid: (i, 0)),
    )
    return pl.pallas_call(
        _grouped_body,
        grid_spec=grid_spec,
        out_shape=jax.ShapeDtypeStruct((_PADT + _T, _DIM), jnp.float32),
        compiler_params=pltpu.CompilerParams(
            dimension_semantics=("arbitrary",)),
    )(eid_all, xs2, xb, wp, W13, W2b, Ws13, bs13, Ws2b, bs2r)


def _sc_combine(out3, p0, p1):
    tok_per = _T // _NW
    chunk = 16
    nchunk = tok_per // chunk
    mesh = plsc.VectorSubcoreMesh(core_axis_name="c", subcore_axis_name="s")

    @functools.partial(
        pl.kernel, mesh=mesh,
        out_type=jax.ShapeDtypeStruct((_T, _DIM), jnp.float32),
        scratch_types=[
            pltpu.VMEM((chunk,), jnp.int32),
            pltpu.VMEM((chunk,), jnp.int32),
            pltpu.VMEM((chunk, _DIM), jnp.float32),
            pltpu.VMEM((chunk, _DIM), jnp.float32),
            pltpu.VMEM((chunk, _DIM), jnp.float32),
            pltpu.VMEM((chunk, _DIM), jnp.float32),
            pltpu.SemaphoreType.DMA,
            pltpu.SemaphoreType.DMA,
            pltpu.SemaphoreType.DMA,
        ],
    )
    def k(out_hbm, p0_hbm, p1_hbm, y_hbm,
          i0_v, i1_v, r0_v, r1_v, rz_v, y_v, s0, s1, s2):
        wid = lax.axis_index("s") * 2 + lax.axis_index("c")

        def body(c, carry):
            base = wid * tok_per + c * chunk
            pltpu.sync_copy(p0_hbm.at[pl.ds(base, chunk)], i0_v)
            pltpu.sync_copy(p1_hbm.at[pl.ds(base, chunk)], i1_v)
            c0 = pltpu.async_copy(out_hbm.at[i0_v], r0_v, s0)
            c1 = pltpu.async_copy(out_hbm.at[i1_v], r1_v, s1)
            cz = pltpu.async_copy(
                out_hbm.at[pl.ds(_ZBASE + base, chunk)], rz_v, s2)
            c0.wait()
            c1.wait()
            cz.wait()
            for r in range(chunk):
                for q in range(64):
                    sl = pl.ds(q * 16, 16)
                    y_v[r, sl] = r0_v[r, sl] + r1_v[r, sl] + rz_v[r, sl]
            pltpu.sync_copy(y_v, y_hbm.at[pl.ds(base, chunk)])
            return carry

        lax.fori_loop(0, nchunk, body, 0)

    return k(out3, p0, p1)


def kernel(x, Wg, W1, W2, W3, Ws1, bs1, Ws2, bs2, Ws3, bs3):
    orig_shape = x.shape
    xf = x.reshape(-1, _DIM)
    bf = jnp.bfloat16
    xb = xf.astype(bf)
    Wgb = Wg.astype(bf)
    W13 = jnp.concatenate([W1, W3], axis=1).astype(bf)
    W2b = W2.astype(bf)
    Ws13 = jnp.concatenate([Ws1, Ws3], axis=0).astype(bf)
    bs13 = jnp.concatenate([bs1, bs3]).reshape(1, 2 * _SHINTER)
    Ws2b = Ws2.astype(bf)
    bs2r = bs2.reshape(1, _DIM)

    am, w01, l = _gate(xb, Wgb)

    e_flat = am.reshape(-1)
    oh = (e_flat[:, None] ==
          jnp.arange(_E, dtype=jnp.int32)[None, :]).astype(jnp.int32)
    cc = jnp.cumsum(oh, axis=0)
    counts = cc[-1]
    rank = jnp.take_along_axis(cc, e_flat[:, None], axis=1)[:, 0] - 1
    pc = ((counts + _TILE - 1) // _TILE) * _TILE
    ends = jnp.cumsum(pc)
    pofs = ends - pc
    p_flat = pofs[e_flat] + rank
    tile_eid = jnp.minimum(
        jnp.searchsorted(ends, jnp.arange(_NRT, dtype=jnp.int32) * _TILE,
                         side='right'),
        _E - 1).astype(jnp.int32)
    eid_all = jnp.concatenate(
        [tile_eid, jnp.full((_NST,), _E, jnp.int32)])
    tok = jnp.arange(_NP, dtype=jnp.int32) // 2
    tok_padded = jnp.zeros((_PADT,), jnp.int32).at[p_flat].set(tok)
    w_padded = jnp.zeros((_PADT,), jnp.float32).at[p_flat].set(
        w01.reshape(-1))
    p01 = p_flat.reshape(_T, 2)

    xs = _sc_gather(xf, tok_padded.reshape(4 * _NW, _PADT // (4 * _NW)))

    out = _grouped(eid_all, xs, xb, w_padded.reshape(_PADT, 1), W13, W2b,
                   Ws13, bs13, Ws2b, bs2r)

    y = _sc_combine(out, p01[:, 0], p01[:, 1])

    return y.reshape(orig_shape), l[0, 0]


# speedup vs baseline: 3.2710x; 2.9021x over previous
import jax
import jax.numpy as jnp
from jax.experimental import pallas as pl
from jax.experimental.pallas import tpu as pltpu

_DIM = 1024
_INTER = 512
_E = 8
_TOPK = 2
_SHINTER = 1024


def _dot_t(a, b, prec=None):
    return jax.lax.dot_general(
        a, b, (((1,), (1,)), ((), ())),
        preferred_element_type=jnp.float32, precision=prec)


def _moe_body(xb_ref, wg_ref, w13_ref, w2_ref,
              ws13_ref, bs13_ref, ws2_ref, bs2_ref,
              y_ref, l_ref, w_scr):
    e = pl.program_id(0)
    T = xb_ref.shape[0]

    @pl.when(e == 0)
    def _gate():
        scores = _dot_t(xb_ref[...], wg_ref[...])
        p = jax.nn.sigmoid(scores)
        iota = jax.lax.broadcasted_iota(jnp.int32, p.shape, 1)
        m1 = jnp.max(p, axis=1, keepdims=True)
        am1 = jnp.min(jnp.where(p == m1, iota, _E), axis=1, keepdims=True)
        p2 = jnp.where(iota == am1, -1.0, p)
        m2 = jnp.max(p2, axis=1, keepdims=True)
        am2 = jnp.min(jnp.where(p2 == m2, iota, _E), axis=1, keepdims=True)
        s = m1 + m2
        w = (jnp.where(iota == am1, m1, 0.0) +
             jnp.where(iota == am2, m2, 0.0)) / s
        w_scr[...] = w
        sel = ((iota == am1) | (iota == am2)).astype(jnp.float32)
        counts = jnp.sum(sel, axis=0, keepdims=True)
        probs = jnp.sum(w, axis=0, keepdims=True)
        f_i = _E * counts / (_TOPK * T)
        p_i = probs / T
        l_ref[...] = jnp.sum(f_i * p_i, axis=1, keepdims=True)

    @pl.when(e < _E)
    def _routed():
        g = _dot_t(xb_ref[...], w13_ref[0])
        h = jax.nn.silu(g[:, :_INTER]) * g[:, _INTER:]
        out = _dot_t(h.astype(jnp.bfloat16), w2_ref[0])
        iota = jax.lax.broadcasted_iota(jnp.int32, (T, _E), 1)
        wtok = jnp.sum(jnp.where(iota == e, w_scr[...], 0.0),
                       axis=1, keepdims=True)
        contrib = out * wtok

        @pl.when(e == 0)
        def _():
            y_ref[...] = contrib

        @pl.when(e > 0)
        def _():
            y_ref[...] += contrib

    @pl.when(e == _E)
    def _shared():
        gs = _dot_t(xb_ref[...], ws13_ref[...]) + bs13_ref[...]
        hs = jax.nn.silu(gs[:, :_SHINTER]) * gs[:, _SHINTER:]
        z = _dot_t(hs.astype(jnp.bfloat16), ws2_ref[...]) + bs2_ref[...]
        y_ref[...] += z


def kernel(x, Wg, W1, W2, W3, Ws1, bs1, Ws2, bs2, Ws3, bs3):
    orig_shape = x.shape
    xf = x.reshape(-1, _DIM)
    T = xf.shape[0]
    bf = jnp.bfloat16
    xb = xf.astype(bf)
    Wgb = Wg.astype(bf)
    W13 = jnp.concatenate([W1, W3], axis=1).astype(bf)
    W2b = W2.astype(bf)
    Ws13 = jnp.concatenate([Ws1, Ws3], axis=0).astype(bf)
    bs13 = jnp.concatenate([bs1, bs3]).reshape(1, 2 * _SHINTER)
    Ws2b = Ws2.astype(bf)
    bs2r = bs2.reshape(1, _DIM)

    const2 = lambda shape: pl.BlockSpec(shape, lambda e: (0, 0))
    expert3 = lambda shape: pl.BlockSpec(
        shape, lambda e: (jnp.minimum(e, _E - 1), 0, 0))

    y, l = pl.pallas_call(
        _moe_body,
        grid=(_E + 1,),
        in_specs=[
            const2((T, _DIM)),
            const2((_E, _DIM)),
            expert3((1, 2 * _INTER, _DIM)),
            expert3((1, _DIM, _INTER)),
            const2((2 * _SHINTER, _DIM)),
            const2((1, 2 * _SHINTER)),
            const2((_DIM, _SHINTER)),
            const2((1, _DIM)),
        ],
        out_specs=[
            const2((T, _DIM)),
            const2((1, 1)),
        ],
        out_shape=[
            jax.ShapeDtypeStruct((T, _DIM), jnp.float32),
            jax.ShapeDtypeStruct((1, 1), jnp.float32),
        ],
        scratch_shapes=[
            pltpu.VMEM((T, _E), jnp.float32),
        ],
        compiler_params=pltpu.CompilerParams(
            dimension_semantics=("arbitrary",)),
    )(xb, Wgb, W13, W2b, Ws13, bs13, Ws2b, bs2r)
    return y.reshape(orig_shape), l[0, 0]


# speedup vs baseline: 4.4797x; 1.3695x over previous
import jax
import jax.numpy as jnp
from jax.experimental import pallas as pl
from jax.experimental.pallas import tpu as pltpu

_DIM = 1024
_INTER = 512
_E = 8
_TOPK = 2
_SHINTER = 1024


def _dot_t(a, b):
    return jax.lax.dot_general(
        a, b, (((1,), (1,)), ((), ())),
        preferred_element_type=jnp.float32)


def _moe_body(x_ref, wg_ref, w1_ref, w3_ref, w2_ref,
              ws1_ref, bs1_ref, ws3_ref, bs3_ref, ws2_ref, bs2_ref,
              y_ref, l_ref, w_scr):
    e = pl.program_id(0)
    T = x_ref.shape[0]

    @pl.when(e == 0)
    def _gate():
        scores = _dot_t(x_ref[...], wg_ref[...])
        p = jax.nn.sigmoid(scores)
        iota = jax.lax.broadcasted_iota(jnp.int32, p.shape, 1)
        m1 = jnp.max(p, axis=1, keepdims=True)
        am1 = jnp.min(jnp.where(p == m1, iota, _E), axis=1, keepdims=True)
        p2 = jnp.where(iota == am1, -1.0, p)
        m2 = jnp.max(p2, axis=1, keepdims=True)
        am2 = jnp.min(jnp.where(p2 == m2, iota, _E), axis=1, keepdims=True)
        s = m1 + m2
        w = (jnp.where(iota == am1, m1, 0.0) +
             jnp.where(iota == am2, m2, 0.0)) / s
        w_scr[...] = w
        sel = ((iota == am1) | (iota == am2)).astype(jnp.float32)
        counts = jnp.sum(sel, axis=0, keepdims=True)
        probs = jnp.sum(w, axis=0, keepdims=True)
        f_i = _E * counts / (_TOPK * T)
        p_i = probs / T
        l_ref[...] = jnp.sum(f_i * p_i, axis=1, keepdims=True)

    @pl.when(e < _E)
    def _routed():
        x = x_ref[...]
        h1 = _dot_t(x, w1_ref[0])
        h3 = _dot_t(x, w3_ref[0])
        h = jax.nn.silu(h1) * h3
        out = _dot_t(h, w2_ref[0])
        iota = jax.lax.broadcasted_iota(jnp.int32, (T, _E), 1)
        wtok = jnp.sum(jnp.where(iota == e, w_scr[...], 0.0),
                       axis=1, keepdims=True)
        contrib = out * wtok

        @pl.when(e == 0)
        def _():
            y_ref[...] = contrib

        @pl.when(e > 0)
        def _():
            y_ref[...] += contrib

    @pl.when(e == _E)
    def _shared():
        x = x_ref[...]
        g1 = _dot_t(x, ws1_ref[...]) + bs1_ref[...]
        g3 = _dot_t(x, ws3_ref[...]) + bs3_ref[...]
        hs = jax.nn.silu(g1) * g3
        z = _dot_t(hs, ws2_ref[...]) + bs2_ref[...]
        y_ref[...] += z


def kernel(x, Wg, W1, W2, W3, Ws1, bs1, Ws2, bs2, Ws3, bs3):
    orig_shape = x.shape
    xf = x.reshape(-1, _DIM)
    T = xf.shape[0]
    bs1r = bs1.reshape(1, _SHINTER)
    bs2r = bs2.reshape(1, _DIM)
    bs3r = bs3.reshape(1, _SHINTER)

    const2 = lambda shape: pl.BlockSpec(shape, lambda e: (0, 0))
    expert3 = lambda shape: pl.BlockSpec(
        shape, lambda e: (jnp.minimum(e, _E - 1), 0, 0))

    y, l = pl.pallas_call(
        _moe_body,
        grid=(_E + 1,),
        in_specs=[
            const2((T, _DIM)),
            const2((_E, _DIM)),
            expert3((1, _INTER, _DIM)),
            expert3((1, _INTER, _DIM)),
            expert3((1, _DIM, _INTER)),
            const2((_SHINTER, _DIM)),
            const2((1, _SHINTER)),
            const2((_SHINTER, _DIM)),
            const2((1, _SHINTER)),
            const2((_DIM, _SHINTER)),
            const2((1, _DIM)),
        ],
        out_specs=[
            const2((T, _DIM)),
            const2((1, 1)),
        ],
        out_shape=[
            jax.ShapeDtypeStruct((T, _DIM), jnp.float32),
            jax.ShapeDtypeStruct((1, 1), jnp.float32),
        ],
        scratch_shapes=[
            pltpu.VMEM((T, _E), jnp.float32),
        ],
        compiler_params=pltpu.CompilerParams(
            dimension_semantics=("arbitrary",)),
    )(xf, Wg, W1, W3, W2, Ws1, bs1r, Ws3, bs3r, Ws2, bs2r)
    return y.reshape(orig_shape), l[0, 0]
